# baseline (speedup 1.0000x reference)
import jax
import jax.numpy as jnp
from jax import lax
from jax.experimental import pallas as pl
from jax.experimental.pallas import tpu as pltpu
from jax.experimental.pallas import tpu_sc as plsc

B, V, D = 4096, 1000, 128
UC, UN, IC, IN = 26, 13, 10, 6
NC, NS, L = 2, 16, 16
NW = NC * NS
ROWS_PER_W = B // NW
NB = 8
NCHUNKS = ROWS_PER_W // NB
IXP = 16
UXW = ROWS_PER_W + NB

O_UIDX = 0
O_IIDX = O_UIDX + B * UC
O_UX = O_IIDX + B * IXP
O_IX = O_UX + UN * (B + NB)
O_UW = O_IX + B * IXP
O_UB = O_UW + UN * D
O_IW = O_UB + UN * D
O_IB = O_IW + IN * D
SIDE_LEN = O_IB + IN * D
W_UW, W_UB, W_IW, W_IB = 0, UN * D, 2 * UN * D, 2 * UN * D + IN * D


def _body(u_tab, i_tab, side,
          u_out, i_out,
          u_stage, i_stage, u_idx_v, i_idx_v, u_x_v, i_x_v, wb_v,
          sem_g, sem_out):
    wid = lax.axis_index("s") * NC + lax.axis_index("c")
    base = wid * ROWS_PER_W

    pltpu.sync_copy(side.at[pl.ds(O_UW, 2 * (UN + IN) * D)], wb_v)
    pltpu.sync_copy(side.at[pl.ds(O_UIDX + wid * UC * ROWS_PER_W,
                                  UC * ROWS_PER_W)], u_idx_v)
    pltpu.sync_copy(side.at[pl.ds(O_IIDX + base * IXP, ROWS_PER_W * IXP)],
                    i_idx_v)
    for n in range(UN):
        pltpu.sync_copy(side.at[pl.ds(O_UX + n * (B + NB) + base, UXW)],
                        u_x_v.at[pl.ds(n * UXW, UXW)])
    pltpu.sync_copy(side.at[pl.ds(O_IX + base * IXP, ROWS_PER_W * IXP)], i_x_v)

    def step(g, carry):
        for s in range(2):
            ci = 2 * g + s
            cb = ci * NB
            b0 = base + cb

            @pl.when(ci >= 2)
            def _():
                pltpu.make_async_copy(
                    u_stage.at[s], u_out.at[:, pl.ds(0, NB)], sem_out.at[s]).wait()
                pltpu.make_async_copy(
                    i_stage.at[s], i_out.at[pl.ds(0, NB)], sem_out.at[s]).wait()

            gathers = []
            for c in range(UC):
                gathers.append(pltpu.async_copy(
                    u_tab.at[u_idx_v.at[pl.ds(c * ROWS_PER_W + cb, NB)]],
                    u_stage.at[s, c], sem_g.at[s]))
            for bi in range(NB):
                gathers.append(pltpu.async_copy(
                    i_tab.at[i_idx_v.at[pl.ds((cb + bi) * IXP, IC)]],
                    i_stage.at[s, bi, pl.ds(0, IC)], sem_g.at[s]))

            for n in range(UN):
                xcol = plsc.bitcast(u_x_v[pl.ds(n * UXW + cb, L)], jnp.float32)
                xv = [xcol[bi] for bi in range(NB)]
                for dd in range(D // L):
                    wv = plsc.bitcast(
                        wb_v[pl.ds(W_UW + n * D + dd * L, L)], jnp.float32)
                    bv = plsc.bitcast(
                        wb_v[pl.ds(W_UB + n * D + dd * L, L)], jnp.float32)
                    sl = pl.ds(dd * L, L)
                    for bi in range(NB):
                        u_stage[s, UC + n, bi, sl] = xv[bi] * wv + bv
            xrows = [plsc.bitcast(i_x_v[pl.ds((cb + bi) * IXP, L)], jnp.float32)
                     for bi in range(NB)]
            for n in range(IN):
                for dd in range(D // L):
                    wv = plsc.bitcast(
                        wb_v[pl.ds(W_IW + n * D + dd * L, L)], jnp.float32)
                    bv = plsc.bitcast(
                        wb_v[pl.ds(W_IB + n * D + dd * L, L)], jnp.float32)
                    sl = pl.ds(dd * L, L)
                    for bi in range(NB):
                        i_stage[s, bi, IC + n, sl] = xrows[bi][n] * wv + bv

            for c in gathers:
                c.wait()

            pltpu.async_copy(u_stage.at[s], u_out.at[:, pl.ds(b0, NB)], sem_out.at[s])
            pltpu.async_copy(i_stage.at[s], i_out.at[pl.ds(b0, NB)], sem_out.at[s])
        return carry

    lax.fori_loop(0, NCHUNKS // 2, step, None)

    for s in range(2):
        pltpu.make_async_copy(
            u_stage.at[s], u_out.at[:, pl.ds(0, NB)], sem_out.at[s]).wait()
        pltpu.make_async_copy(
            i_stage.at[s], i_out.at[pl.ds(0, NB)], sem_out.at[s]).wait()


@jax.jit
def _encode(u_tab, i_tab, side):
    mesh = plsc.VectorSubcoreMesh(core_axis_name="c", subcore_axis_name="s")
    f = pl.kernel(
        _body,
        out_type=(
            jax.ShapeDtypeStruct((UC + UN, B, D), jnp.float32),
            jax.ShapeDtypeStruct((B, IC + IN, D), jnp.float32),
        ),
        mesh=mesh,
        compiler_params=pltpu.CompilerParams(needs_layout_passes=False),
        scratch_types=[
            pltpu.VMEM((2, UC + UN, NB, D), jnp.float32),
            pltpu.VMEM((2, NB, IC + IN, D), jnp.float32),
            pltpu.VMEM((UC * ROWS_PER_W,), jnp.int32),
            pltpu.VMEM((ROWS_PER_W * IXP,), jnp.int32),
            pltpu.VMEM((UN * UXW,), jnp.int32),
            pltpu.VMEM((ROWS_PER_W * IXP,), jnp.int32),
            pltpu.VMEM((2 * (UN + IN) * D,), jnp.int32),
            pltpu.SemaphoreType.DMA((2,)),
            pltpu.SemaphoreType.DMA((2,)),
        ],
    )
    return f(u_tab, i_tab, side)


def kernel(users_cat, users_num, items_cat, items_num,
           users_cat_emb, users_num_w, users_num_b,
           items_cat_emb, items_num_w, items_num_b):
    f2i = lambda a: jax.lax.bitcast_convert_type(a, jnp.int32)
    u_idxt = (users_cat.astype(jnp.int32).T
              + (jnp.arange(UC, dtype=jnp.int32) * V)[:, None]
              ).reshape(UC, NW, ROWS_PER_W).transpose(1, 0, 2).reshape(-1)
    i_idxp = jnp.pad(
        items_cat.astype(jnp.int32)
        + (jnp.arange(IC, dtype=jnp.int32) * V)[None, :],
        ((0, 0), (0, IXP - IC))).reshape(-1)
    u_xt = f2i(jnp.pad(users_num.T, ((0, 0), (0, NB))).reshape(-1))
    i_xp = f2i(jnp.pad(items_num, ((0, 0), (0, IXP - IN))).reshape(-1))
    side = jnp.concatenate([
        u_idxt, i_idxp, u_xt, i_xp,
        f2i(users_num_w.reshape(-1)), f2i(users_num_b.reshape(-1)),
        f2i(items_num_w.reshape(-1)), f2i(items_num_b.reshape(-1)),
    ])
    u_tab = users_cat_emb.reshape(UC * V, D)
    i_tab = items_cat_emb.reshape(IC * V, D)
    u_out_t, i_out = _encode(u_tab, i_tab, side)
    return (jnp.transpose(u_out_t, (1, 0, 2)), i_out)


# speedup vs baseline: 1.0594x; 1.0594x over previous
import jax
import jax.numpy as jnp
from jax import lax
from jax.experimental import pallas as pl
from jax.experimental.pallas import tpu as pltpu
from jax.experimental.pallas import tpu_sc as plsc

B, V, D = 4096, 1000, 128
UC, UN, IC, IN = 26, 13, 10, 6
NC, NS, L = 2, 16, 16
NW = NC * NS
ROWS_PER_W = B // NW
NB = 8
NCHUNKS = ROWS_PER_W // NB
IXP = 16
UXW = ROWS_PER_W + NB


def _body(u_tab, i_tab, u_idxt, i_idxp, u_xt, i_xp, u_w, u_b, i_w, i_b,
          u_out, i_out,
          u_stage, i_stage, u_idx_v, i_idx_v, u_x_v, i_x_v,
          u_w_v, u_b_v, i_w_v, i_b_v,
          sem_g, sem_out):
    wid = lax.axis_index("s") * NC + lax.axis_index("c")
    base = wid * ROWS_PER_W

    pltpu.sync_copy(u_w, u_w_v)
    pltpu.sync_copy(u_b, u_b_v)
    pltpu.sync_copy(i_w, i_w_v)
    pltpu.sync_copy(i_b, i_b_v)
    pltpu.sync_copy(u_idxt.at[pl.ds(wid * UC * ROWS_PER_W, UC * ROWS_PER_W)],
                    u_idx_v)
    pltpu.sync_copy(i_idxp.at[pl.ds(base * IXP, ROWS_PER_W * IXP)], i_idx_v)
    for n in range(UN):
        pltpu.sync_copy(u_xt.at[pl.ds(n * (B + NB) + base, UXW)],
                        u_x_v.at[pl.ds(n * UXW, UXW)])
    pltpu.sync_copy(i_xp.at[pl.ds(base * IXP, ROWS_PER_W * IXP)], i_x_v)

    def step(g, carry):
        for s in range(2):
            ci = 2 * g + s
            cb = ci * NB
            b0 = base + cb

            @pl.when(ci >= 2)
            def _():
                pltpu.make_async_copy(
                    u_stage.at[s], u_out.at[:, pl.ds(0, NB)], sem_out.at[s]).wait()
                pltpu.make_async_copy(
                    i_stage.at[s], i_out.at[pl.ds(0, NB)], sem_out.at[s]).wait()

            gathers = []
            for c in range(UC):
                gathers.append(pltpu.async_copy(
                    u_tab.at[u_idx_v.at[pl.ds(c * ROWS_PER_W + cb, NB)]],
                    u_stage.at[s, c], sem_g.at[s]))
            for bi in range(NB):
                gathers.append(pltpu.async_copy(
                    i_tab.at[i_idx_v.at[pl.ds((cb + bi) * IXP, IC)]],
                    i_stage.at[s, bi, pl.ds(0, IC)], sem_g.at[s]))

            for n in range(UN):
                xcol = u_x_v[pl.ds(n * UXW + cb, L)]
                xv = [xcol[bi] for bi in range(NB)]
                for dd in range(D // L):
                    sl = pl.ds(dd * L, L)
                    wv = u_w_v[n, sl]
                    bv = u_b_v[n, sl]
                    for bi in range(NB):
                        u_stage[s, UC + n, bi, sl] = xv[bi] * wv + bv
            xrows = [i_x_v[pl.ds((cb + bi) * IXP, L)] for bi in range(NB)]
            for n in range(IN):
                for dd in range(D // L):
                    sl = pl.ds(dd * L, L)
                    wv = i_w_v[n, sl]
                    bv = i_b_v[n, sl]
                    for bi in range(NB):
                        i_stage[s, bi, IC + n, sl] = xrows[bi][n] * wv + bv

            for c in gathers:
                c.wait()

            pltpu.async_copy(u_stage.at[s], u_out.at[:, pl.ds(b0, NB)], sem_out.at[s])
            pltpu.async_copy(i_stage.at[s], i_out.at[pl.ds(b0, NB)], sem_out.at[s])
        return carry

    lax.fori_loop(0, NCHUNKS // 2, step, None)

    for s in range(2):
        pltpu.make_async_copy(
            u_stage.at[s], u_out.at[:, pl.ds(0, NB)], sem_out.at[s]).wait()
        pltpu.make_async_copy(
            i_stage.at[s], i_out.at[pl.ds(0, NB)], sem_out.at[s]).wait()


@jax.jit
def _encode(u_tab, i_tab, u_idxt, i_idxp, u_xt, i_xp, u_w, u_b, i_w, i_b):
    mesh = plsc.VectorSubcoreMesh(core_axis_name="c", subcore_axis_name="s")
    f = pl.kernel(
        _body,
        out_type=(
            jax.ShapeDtypeStruct((UC + UN, B, D), jnp.float32),
            jax.ShapeDtypeStruct((B, IC + IN, D), jnp.float32),
        ),
        mesh=mesh,
        scratch_types=[
            pltpu.VMEM((2, UC + UN, NB, D), jnp.float32),
            pltpu.VMEM((2, NB, IC + IN, D), jnp.float32),
            pltpu.VMEM((UC * ROWS_PER_W,), jnp.int32),
            pltpu.VMEM((ROWS_PER_W * IXP,), jnp.int32),
            pltpu.VMEM((UN * UXW,), jnp.float32),
            pltpu.VMEM((ROWS_PER_W * IXP,), jnp.float32),
            pltpu.VMEM((UN, D), jnp.float32),
            pltpu.VMEM((UN, D), jnp.float32),
            pltpu.VMEM((IN, D), jnp.float32),
            pltpu.VMEM((IN, D), jnp.float32),
            pltpu.SemaphoreType.DMA((2,)),
            pltpu.SemaphoreType.DMA((2,)),
        ],
    )
    return f(u_tab, i_tab, u_idxt, i_idxp, u_xt, i_xp, u_w, u_b, i_w, i_b)


def kernel(users_cat, users_num, items_cat, items_num,
           users_cat_emb, users_num_w, users_num_b,
           items_cat_emb, items_num_w, items_num_b):
    u_idxt = (users_cat.astype(jnp.int32).T
              + (jnp.arange(UC, dtype=jnp.int32) * V)[:, None]
              ).reshape(UC, NW, ROWS_PER_W).transpose(1, 0, 2).reshape(-1)
    i_idxp = jnp.pad(
        items_cat.astype(jnp.int32)
        + (jnp.arange(IC, dtype=jnp.int32) * V)[None, :],
        ((0, 0), (0, IXP - IC))).reshape(-1)
    u_xt = jnp.pad(users_num.T, ((0, 0), (0, NB))).reshape(-1)
    i_xp = jnp.pad(items_num, ((0, 0), (0, IXP - IN))).reshape(-1)
    u_tab = users_cat_emb.reshape(UC * V, D)
    i_tab = items_cat_emb.reshape(IC * V, D)
    u_out_t, i_out = _encode(u_tab, i_tab, u_idxt, i_idxp, u_xt, i_xp,
                             users_num_w, users_num_b,
                             items_num_w, items_num_b)
    return (jnp.transpose(u_out_t, (1, 0, 2)), i_out)


# speedup vs baseline: 1.0676x; 1.0078x over previous
import jax
import jax.numpy as jnp
from jax import lax
from jax.experimental import pallas as pl
from jax.experimental.pallas import tpu as pltpu
from jax.experimental.pallas import tpu_sc as plsc

B, V, D = 4096, 1000, 128
UC, UN, IC, IN = 26, 13, 10, 6
NC, NS, L = 2, 16, 16
NW = NC * NS
ROWS_PER_W = B // NW
NB = 8
NCHUNKS = ROWS_PER_W // NB
IXP = 16
UXW = ROWS_PER_W + NB


def _body(u_tab, i_tab, u_idxt, i_idxp, u_xt, i_xp, u_w, u_b, i_w, i_b,
          u_out, i_out,
          u_stage, i_stage, u_idx_v, i_idx_v, u_x_v, i_x_v,
          u_w_v, u_b_v, i_w_v, i_b_v,
          sem_g, sem_out):
    wid = lax.axis_index("s") * NC + lax.axis_index("c")
    base = wid * ROWS_PER_W

    pltpu.sync_copy(u_w, u_w_v)
    pltpu.sync_copy(u_b, u_b_v)
    pltpu.sync_copy(i_w, i_w_v)
    pltpu.sync_copy(i_b, i_b_v)
    pltpu.sync_copy(u_idxt.at[pl.ds(wid * UC * ROWS_PER_W, UC * ROWS_PER_W)],
                    u_idx_v)
    pltpu.sync_copy(i_idxp.at[pl.ds(base * IXP, ROWS_PER_W * IXP)], i_idx_v)
    for n in range(UN):
        pltpu.sync_copy(u_xt.at[pl.ds(n * (B + NB) + base, UXW)],
                        u_x_v.at[pl.ds(n * UXW, UXW)])
    pltpu.sync_copy(i_xp.at[pl.ds(base * IXP, ROWS_PER_W * IXP)], i_x_v)

    def step(g, carry):
        for s in range(2):
            ci = 2 * g + s
            cb = ci * NB
            b0 = base + cb

            @pl.when(ci >= 2)
            def _():
                pltpu.make_async_copy(
                    u_stage.at[s], u_out.at[:, pl.ds(0, NB)], sem_out.at[s]).wait()
                pltpu.make_async_copy(
                    i_stage.at[s], i_out.at[pl.ds(0, NB)], sem_out.at[s]).wait()

            gathers = []
            for c in range(UC):
                gathers.append(pltpu.async_copy(
                    u_tab.at[u_idx_v.at[pl.ds(c * ROWS_PER_W + cb, NB)]],
                    u_stage.at[s, c], sem_g.at[s]))
            for bi in range(NB):
                gathers.append(pltpu.async_copy(
                    i_tab.at[i_idx_v.at[pl.ds((cb + bi) * IXP, IC)]],
                    i_stage.at[s, bi, pl.ds(0, IC)], sem_g.at[s]))

            for n in range(UN):
                xcol = u_x_v[pl.ds(n * UXW + cb, L)]
                xv = [xcol[bi] for bi in range(NB)]
                for dd in range(D // L):
                    sl = pl.ds(dd * L, L)
                    wv = u_w_v[n, sl]
                    bv = u_b_v[n, sl]
                    for bi in range(NB):
                        u_stage[s, UC + n, bi, sl] = xv[bi] * wv + bv
            xrows = [i_x_v[pl.ds((cb + bi) * IXP, L)] for bi in range(NB)]
            for n in range(IN):
                for dd in range(D // L):
                    sl = pl.ds(dd * L, L)
                    wv = i_w_v[n, sl]
                    bv = i_b_v[n, sl]
                    for bi in range(NB):
                        i_stage[s, bi, IC + n, sl] = xrows[bi][n] * wv + bv

            for c in gathers[:UC]:
                c.wait()
            pltpu.async_copy(u_stage.at[s], u_out.at[:, pl.ds(b0, NB)], sem_out.at[s])
            for c in gathers[UC:]:
                c.wait()
            pltpu.async_copy(i_stage.at[s], i_out.at[pl.ds(b0, NB)], sem_out.at[s])
        return carry

    lax.fori_loop(0, NCHUNKS // 2, step, None)

    for s in range(2):
        pltpu.make_async_copy(
            u_stage.at[s], u_out.at[:, pl.ds(0, NB)], sem_out.at[s]).wait()
        pltpu.make_async_copy(
            i_stage.at[s], i_out.at[pl.ds(0, NB)], sem_out.at[s]).wait()


@jax.jit
def _encode(u_tab, i_tab, u_idxt, i_idxp, u_xt, i_xp, u_w, u_b, i_w, i_b):
    mesh = plsc.VectorSubcoreMesh(core_axis_name="c", subcore_axis_name="s")
    f = pl.kernel(
        _body,
        out_type=(
            jax.ShapeDtypeStruct((UC + UN, B, D), jnp.float32),
            jax.ShapeDtypeStruct((B, IC + IN, D), jnp.float32),
        ),
        mesh=mesh,
        scratch_types=[
            pltpu.VMEM((2, UC + UN, NB, D), jnp.float32),
            pltpu.VMEM((2, NB, IC + IN, D), jnp.float32),
            pltpu.VMEM((UC * ROWS_PER_W,), jnp.int32),
            pltpu.VMEM((ROWS_PER_W * IXP,), jnp.int32),
            pltpu.VMEM((UN * UXW,), jnp.float32),
            pltpu.VMEM((ROWS_PER_W * IXP,), jnp.float32),
            pltpu.VMEM((UN, D), jnp.float32),
            pltpu.VMEM((UN, D), jnp.float32),
            pltpu.VMEM((IN, D), jnp.float32),
            pltpu.VMEM((IN, D), jnp.float32),
            pltpu.SemaphoreType.DMA((2,)),
            pltpu.SemaphoreType.DMA((2,)),
        ],
    )
    return f(u_tab, i_tab, u_idxt, i_idxp, u_xt, i_xp, u_w, u_b, i_w, i_b)


def kernel(users_cat, users_num, items_cat, items_num,
           users_cat_emb, users_num_w, users_num_b,
           items_cat_emb, items_num_w, items_num_b):
    u_idxt = (users_cat.astype(jnp.int32).T
              + (jnp.arange(UC, dtype=jnp.int32) * V)[:, None]
              ).reshape(UC, NW, ROWS_PER_W).transpose(1, 0, 2).reshape(-1)
    i_idxp = jnp.pad(
        items_cat.astype(jnp.int32)
        + (jnp.arange(IC, dtype=jnp.int32) * V)[None, :],
        ((0, 0), (0, IXP - IC))).reshape(-1)
    u_xt = jnp.pad(users_num.T, ((0, 0), (0, NB))).reshape(-1)
    i_xp = jnp.pad(items_num, ((0, 0), (0, IXP - IN))).reshape(-1)
    u_tab = users_cat_emb.reshape(UC * V, D)
    i_tab = items_cat_emb.reshape(IC * V, D)
    u_out_t, i_out = _encode(u_tab, i_tab, u_idxt, i_idxp, u_xt, i_xp,
                             users_num_w, users_num_b,
                             items_num_w, items_num_b)
    return (jnp.transpose(u_out_t, (1, 0, 2)), i_out)
